# Initial kernel scaffold; baseline (speedup 1.0000x reference)
import jax
import jax.numpy as jnp
from jax.experimental import pallas as pl


def kernel(action1, actions2, emb1, emb2):
    raise NotImplementedError("write your pallas kernel here")


# speedup vs baseline: 3.1503x; 3.1503x over previous
import functools

import jax
import jax.numpy as jnp
from jax import lax
from jax.experimental import pallas as pl
from jax.experimental.pallas import tpu as pltpu
from jax.experimental.pallas import tpu_sc as plsc

D = 256
SEQ = 20
N = 4096
EPS = 1e-6
NC, NS = 2, 16
NW = NC * NS
CPW = N // NW
CHUNK = 8
NSTREAM = 2
SPC = CHUNK // NSTREAM * SEQ
IPC = CHUNK * SEQ
NCHUNK = CPW // CHUNK
NCOL = D // 16
A1PAD = 32


def _sc_body(a1i_hbm, a2i_hbm, emb1_hbm, emb2_hbm, num_hbm, den_hbm,
             idx_v, rows0_v, rows1_v, a1i_v, a1rows_v, numv, denv,
             sem0, sem1, sema):
  w = lax.axis_index("s") * NC + lax.axis_index("c")

  lane_iota = lax.iota(jnp.int32, 16)
  zeros = tuple(jnp.zeros((16,), jnp.float32) for _ in range(NCOL))

  def allsum(v):
    for d in (8, 4, 2, 1):
      v = v + v.at[lane_iota ^ d].get(mode="promise_in_bounds")
    return v

  pltpu.sync_copy(a1i_hbm, a1i_v)
  pltpu.async_copy(emb1_hbm.at[a1i_v], a1rows_v, sema).wait()

  def a1_body(t, accs):
    return tuple(accs[k] + a1rows_v[t, pl.ds(16 * k, 16)] for k in range(NCOL))

  s1 = lax.fori_loop(0, SEQ, a1_body, zeros)
  d1v = s1[0] * s1[0]
  for k in range(1, NCOL):
    d1v = d1v + s1[k] * s1[k]
  den1 = allsum(d1v)

  pltpu.sync_copy(a2i_hbm.at[w], idx_v)

  def start_gather(ci, rows, sem):
    for h in range(NSTREAM):
      pltpu.async_copy(
          emb2_hbm.at[idx_v.at[ci, h]],
          rows.at[pl.ds(h * SPC, SPC)], sem)

  def wait_gather(ci, rows, sem):
    for h in range(NSTREAM):
      pltpu.make_async_copy(
          emb2_hbm.at[idx_v.at[ci, h]],
          rows.at[pl.ds(h * SPC, SPC)], sem).wait()

  start_gather(0, rows0_v, sem0)
  start_gather(1, rows1_v, sem1)

  def outer(i, _):
    ci0 = 2 * i
    num_acc = jnp.zeros((16,), jnp.float32)
    den_acc = jnp.zeros((16,), jnp.float32)
    for b, (rows, sem) in enumerate(((rows0_v, sem0), (rows1_v, sem1))):
      ci = ci0 + b
      wait_gather(ci, rows, sem)
      for j in range(CHUNK):
        base = j * SEQ

        def seq_body(t, accs):
          return tuple(
              accs[k] + rows[base + t, pl.ds(16 * k, 16)] for k in range(NCOL))

        s2 = lax.fori_loop(0, SEQ, seq_body, zeros)
        nv = s2[0] * s1[0]
        dv = s2[0] * s2[0]
        for k in range(1, NCOL):
          nv = nv + s2[k] * s1[k]
          dv = dv + s2[k] * s2[k]
        lane = b * CHUNK + j
        num_acc = jnp.where(lane_iota == lane, allsum(nv), num_acc)
        den_acc = jnp.where(lane_iota == lane, allsum(dv) * den1, den_acc)

      @pl.when(ci + 2 < NCHUNK)
      def _():
        start_gather(ci + 2, rows, sem)

    numv[pl.ds(16 * i, 16)] = num_acc
    denv[pl.ds(16 * i, 16)] = den_acc
    return _

  lax.fori_loop(0, NCHUNK // 2, outer, None)

  pltpu.sync_copy(numv, num_hbm.at[pl.ds(w * CPW, CPW)])
  pltpu.sync_copy(denv, den_hbm.at[pl.ds(w * CPW, CPW)])


_sc_kernel = functools.partial(
    pl.kernel,
    mesh=plsc.VectorSubcoreMesh(core_axis_name="c", subcore_axis_name="s"),
    out_type=[
        jax.ShapeDtypeStruct((N,), jnp.float32),
        jax.ShapeDtypeStruct((N,), jnp.float32),
    ],
    scratch_types=[
        pltpu.VMEM((NCHUNK, NSTREAM, SPC), jnp.int32),
        pltpu.VMEM((IPC, D), jnp.float32),
        pltpu.VMEM((IPC, D), jnp.float32),
        pltpu.VMEM((A1PAD,), jnp.int32),
        pltpu.VMEM((A1PAD, D), jnp.float32),
        pltpu.VMEM((CPW,), jnp.float32),
        pltpu.VMEM((CPW,), jnp.float32),
        pltpu.SemaphoreType.DMA,
        pltpu.SemaphoreType.DMA,
        pltpu.SemaphoreType.DMA,
    ],
)(_sc_body)


def _fin_body(num_ref, den_ref, out_ref):
  out_ref[...] = num_ref[...] / jnp.maximum(
      jnp.sqrt(den_ref[...]), 400.0 * EPS)


_fin = pl.pallas_call(
    _fin_body,
    out_shape=jax.ShapeDtypeStruct((32, 128), jnp.float32),
)


def kernel(action1, actions2, emb1, emb2):
  a1p = jnp.concatenate(
      [action1, jnp.zeros((A1PAD - SEQ,), jnp.int32)])
  a2r = actions2.reshape(NW, NCHUNK, NSTREAM, SPC)
  num, den = _sc_kernel(a1p, a2r, emb1, emb2)
  cos = _fin(num.reshape(32, 128), den.reshape(32, 128))
  return cos.reshape(N)
